# baseline (speedup 1.0000x reference)
import jax
import jax.numpy as jnp
from jax import lax
from jax.experimental import pallas as pl
from jax.experimental.pallas import tpu as pltpu
from jax.experimental.pallas import tpu_sc as plsc

_ROWS = 16384
_COLS = 1000
_NC = 2
_NS = 16
_NW = _NC * _NS
_ROWS_PER_W = _ROWS // _NW
_CHUNK = 32
_NCHUNKS = _ROWS_PER_W // _CHUNK
_L = 16
_CHUNK_ELEMS = _CHUNK * _COLS
_IDXB = 128
_NIDXB = _ROWS_PER_W // _IDXB


def _sc_body(sav_hbm, act_hbm, qp_hbm, out_hbm,
             buf0, buf1, act_v, qp_v,
             idx0, idx1, idx2, idx3,
             sem_in0, sem_in1, sem_out0, sem_out1, sem_sc):
    wid = lax.axis_index("s") * _NC + lax.axis_index("c")
    base = wid * _ROWS_PER_W

    pltpu.sync_copy(act_hbm.at[pl.ds(base, _ROWS_PER_W)], act_v)
    pltpu.sync_copy(qp_hbm.at[pl.ds(base, _ROWS_PER_W)], qp_v)

    bufs = (buf0, buf1)
    sems_in = (sem_in0, sem_in1)
    sems_out = (sem_out0, sem_out1)
    idx_bufs = (idx0, idx1, idx2, idx3)

    def start_in(c):
        return pltpu.async_copy(
            sav_hbm.at[pl.ds((base + c * _CHUNK) * _COLS, _CHUNK_ELEMS)],
            bufs[c % 2], sems_in[c % 2])

    def start_out(c):
        return pltpu.async_copy(
            bufs[c % 2],
            out_hbm.at[pl.ds((base + c * _CHUNK) * _COLS, _CHUNK_ELEMS)],
            sems_out[c % 2])

    lane = lax.iota(jnp.int32, _L)

    d_in = [None, None]
    d_out = [None, None]
    d_in[0] = start_in(0)
    for g in range(_ROWS_PER_W // _L):
        cols = act_v[pl.ds(g * _L, _L)]
        flat = (lane + (base + g * _L)) * _COLS + cols
        j, k = divmod(g, _IDXB // _L)
        idx_bufs[j][pl.ds(k * _L, _L)] = flat

    for c in range(_NCHUNKS):
        b = c % 2
        nb = (c + 1) % 2
        if c + 1 < _NCHUNKS:
            if c >= 1:
                d_out[nb].wait()
            d_in[nb] = start_in(c + 1)
        d_in[b].wait()
        d_out[b] = start_out(c)
    d_out[0].wait()
    d_out[1].wait()

    d_sc = []
    for j in range(_NIDXB):
        d_sc.append(pltpu.async_copy(
            qp_v.at[pl.ds(j * _IDXB, _IDXB)],
            out_hbm.at[idx_bufs[j]], sem_sc))
    for d in d_sc:
        d.wait()


def kernel(state_action_values, action, q_prime):
    sav_flat = state_action_values.reshape(_ROWS * _COLS)
    act_flat = action.reshape(_ROWS)
    mesh = plsc.VectorSubcoreMesh(core_axis_name="c", subcore_axis_name="s")
    f = pl.kernel(
        _sc_body,
        mesh=mesh,
        out_type=jax.ShapeDtypeStruct((_ROWS * _COLS,), jnp.float32),
        scratch_types=[
            pltpu.VMEM((_CHUNK_ELEMS,), jnp.float32),
            pltpu.VMEM((_CHUNK_ELEMS,), jnp.float32),
            pltpu.VMEM((_ROWS_PER_W,), jnp.int32),
            pltpu.VMEM((_ROWS_PER_W,), jnp.float32),
            pltpu.VMEM((_IDXB,), jnp.int32),
            pltpu.VMEM((_IDXB,), jnp.int32),
            pltpu.VMEM((_IDXB,), jnp.int32),
            pltpu.VMEM((_IDXB,), jnp.int32),
            pltpu.SemaphoreType.DMA,
            pltpu.SemaphoreType.DMA,
            pltpu.SemaphoreType.DMA,
            pltpu.SemaphoreType.DMA,
            pltpu.SemaphoreType.DMA,
        ],
    )
    out_flat = f(sav_flat, act_flat, q_prime)
    return out_flat.reshape(_ROWS, _COLS)


# speedup vs baseline: 1.8621x; 1.8621x over previous
import jax
import jax.numpy as jnp
from jax.experimental import pallas as pl

_ROWS = 16384
_COLS = 1000
_BLOCK_ROWS = 2048


def _body(sav_ref, act_ref, qp_ref, out_ref):
    cols = jax.lax.broadcasted_iota(jnp.int32, sav_ref.shape, 1)
    out_ref[...] = jnp.where(cols == act_ref[...], qp_ref[...], sav_ref[...])


def kernel(state_action_values, action, q_prime):
    qp2 = q_prime.reshape(_ROWS, 1)
    grid = (_ROWS // _BLOCK_ROWS,)
    return pl.pallas_call(
        _body,
        grid=grid,
        in_specs=[
            pl.BlockSpec((_BLOCK_ROWS, _COLS), lambda i: (i, 0)),
            pl.BlockSpec((_BLOCK_ROWS, 1), lambda i: (i, 0)),
            pl.BlockSpec((_BLOCK_ROWS, 1), lambda i: (i, 0)),
        ],
        out_specs=pl.BlockSpec((_BLOCK_ROWS, _COLS), lambda i: (i, 0)),
        out_shape=jax.ShapeDtypeStruct((_ROWS, _COLS), jnp.float32),
    )(state_action_values, action, qp2)
